# Initial kernel scaffold; baseline (speedup 1.0000x reference)
import jax
import jax.numpy as jnp
from jax.experimental import pallas as pl


def kernel(x, edge_index, W, b, velocity_factors, attn_w, out_W, out_b):
    raise NotImplementedError("write your pallas kernel here")


# speedup vs baseline: 2.9478x; 2.9478x over previous
import functools

import jax
import jax.numpy as jnp
from jax import lax
from jax.experimental import pallas as pl
from jax.experimental.pallas import tpu as pltpu, tpu_sc as plsc

NC = 2
NS = 16
L = 16
NW = NC * NS


def _rsqrt16(s):
    i = lax.bitcast_convert_type(s, jnp.int32)
    i = jnp.int32(0x5F3759DF) - lax.shift_right_arithmetic(i, 1)
    y = lax.bitcast_convert_type(i, jnp.float32)
    for _ in range(3):
        y = y * (1.5 - 0.5 * s * y * y)
    return y


def _full16(v):
    return jnp.full((L,), v, jnp.int32)


def _slice_rows(n):
    return ((n + NS - 1) // NS + 7) // 8 * 8


def _linear_body(x_ref, w_ref, b_ref, o_ref):
    o_ref[...] = lax.dot_general(
        x_ref[...], w_ref[...], (((1,), (1,)), ((), ())),
        preferred_element_type=jnp.float32) + b_ref[...]


def _tc_linear(x, W, b2):
    N, F = x.shape
    H = W.shape[0]
    R = 1000
    return pl.pallas_call(
        _linear_body,
        grid=(N // R,),
        in_specs=[
            pl.BlockSpec((R, F), lambda i: (i, 0)),
            pl.BlockSpec((H, F), lambda i: (0, 0)),
            pl.BlockSpec((1, H), lambda i: (0, 0)),
        ],
        out_specs=pl.BlockSpec((R, H), lambda i: (i, 0)),
        out_shape=jax.ShapeDtypeStruct((N, H), jnp.float32),
    )(x, W, b2)


def _edge_scalars(pos4, src, dst):
    N = pos4.shape[0] // 4
    E = src.shape[0]
    ET = E // NW
    mesh = plsc.VectorSubcoreMesh(core_axis_name="c", subcore_axis_name="s")

    @functools.partial(
        pl.kernel,
        out_type=[
            jax.ShapeDtypeStruct((E,), jnp.float32),
            jax.ShapeDtypeStruct((E,), jnp.float32),
            jax.ShapeDtypeStruct((NW * L,), jnp.float32),
        ],
        mesh=mesh,
        compiler_params=pltpu.CompilerParams(needs_layout_passes=False),
        scratch_types=[
            pltpu.VMEM((N * 4,), jnp.float32),
            pltpu.VMEM((ET,), jnp.int32),
            pltpu.VMEM((ET,), jnp.int32),
            pltpu.VMEM((ET,), jnp.float32),
            pltpu.VMEM((ET,), jnp.float32),
            pltpu.VMEM((L,), jnp.float32),
        ],
    )
    def k(pos_hbm, src_hbm, dst_hbm, dist_hbm, dirs_hbm, max_hbm,
          pos_v, s_v, d_v, dist_v, dirs_v, m_v):
        wid = lax.axis_index("s") * NC + lax.axis_index("c")
        base = wid * ET
        pltpu.sync_copy(pos_hbm, pos_v)
        pltpu.sync_copy(src_hbm.at[pl.ds(base, ET)], s_v)
        pltpu.sync_copy(dst_hbm.at[pl.ds(base, ET)], d_v)

        def body(i, mx):
            s16 = s_v[pl.ds(i * L, L)] * 4
            d16 = d_v[pl.ds(i * L, L)] * 4
            dx = (plsc.load_gather(pos_v, [s16])
                  - plsc.load_gather(pos_v, [d16]))
            dy = (plsc.load_gather(pos_v, [s16 + 1])
                  - plsc.load_gather(pos_v, [d16 + 1]))
            dz = (plsc.load_gather(pos_v, [s16 + 2])
                  - plsc.load_gather(pos_v, [d16 + 2]))
            ss = dx * dx + dy * dy + dz * dz
            sm = jnp.maximum(ss, 1e-30)
            dist = ss * _rsqrt16(sm)
            dirs = (dx + dy + dz) / (dist + 1e-8)
            dist_v[pl.ds(i * L, L)] = dist
            dirs_v[pl.ds(i * L, L)] = dirs
            return jnp.maximum(mx, dist)

        mx = lax.fori_loop(0, ET // L, body, jnp.zeros((L,), jnp.float32))
        m_v[...] = mx
        pltpu.sync_copy(dist_v, dist_hbm.at[pl.ds(base, ET)])
        pltpu.sync_copy(dirs_v, dirs_hbm.at[pl.ds(base, ET)])
        pltpu.sync_copy(m_v, max_hbm.at[pl.ds(wid * L, L)])

    return k(pos4, src, dst)


def _accumulate(t, src, dst, dist, dirs, maxes, vf16, zrow, nobs):
    N, H = t.shape
    E = src.shape[0]
    ET = E // NW
    CH = 80
    NCHUNK = ET // CH
    SL = _slice_rows(N)
    NP = SL * NS
    mesh = plsc.VectorSubcoreMesh(core_axis_name="c", subcore_axis_name="s")

    @functools.partial(
        pl.kernel,
        out_type=jax.ShapeDtypeStruct((NC * nobs, NP, H), jnp.float32),
        mesh=mesh,
        compiler_params=pltpu.CompilerParams(needs_layout_passes=False),
        scratch_types=[
            pltpu.VMEM_SHARED((NP, H), jnp.float32),
            pltpu.VMEM((CH, H), jnp.float32),
            pltpu.VMEM((CH,), jnp.int32),
            pltpu.VMEM((CH,), jnp.int32),
            pltpu.VMEM((CH,), jnp.float32),
            pltpu.VMEM((CH,), jnp.float32),
            pltpu.VMEM((CH,), jnp.float32),
            pltpu.VMEM((NW * L,), jnp.float32),
            pltpu.VMEM((L,), jnp.float32),
            pltpu.SemaphoreType.DMA,
        ],
    )
    def k(t_hbm, src_hbm, dst_hbm, dist_hbm, dirs_hbm, max_hbm, vf_hbm,
          z_hbm, part_hbm,
          acc, rows, s_c, d_c, di_c, dr_c, w_c, mx_v, vf_v, sem):
        cid = lax.axis_index("c")
        sid = lax.axis_index("s")
        wid = sid * NC + cid
        base = wid * ET

        pltpu.sync_copy(max_hbm, mx_v)
        pltpu.sync_copy(vf_hbm, vf_v)
        m = jnp.zeros((L,), jnp.float32)
        for j in range(NW):
            m = jnp.maximum(m, mx_v[pl.ds(j * L, L)])
        dmax16 = jnp.full((L,), jnp.max(m, axis=0), jnp.float32)
        inv_dmax = 1.0 / dmax16

        vf_all = vf_v[...]
        lane = lax.iota(jnp.int32, L)
        for o in range(nobs):
            vfo = jnp.full(
                (L,),
                jnp.sum(jnp.where(lane == o, vf_all, 0.0), axis=0),
                jnp.float32)
            pltpu.sync_copy(z_hbm, acc.at[pl.ds(sid * SL, SL)])
            plsc.subcore_barrier()

            def chunk(i, _):
                cb = base + i * CH
                pltpu.sync_copy(src_hbm.at[pl.ds(cb, CH)], s_c)
                pltpu.sync_copy(dst_hbm.at[pl.ds(cb, CH)], d_c)
                pltpu.sync_copy(dist_hbm.at[pl.ds(cb, CH)], di_c)
                pltpu.sync_copy(dirs_hbm.at[pl.ds(cb, CH)], dr_c)
                pltpu.async_copy(t_hbm.at[s_c], rows, sem).wait()
                for g in range(CH // L):
                    dist16 = di_c[pl.ds(g * L, L)]
                    dirs16 = dr_c[pl.ds(g * L, L)]
                    v = vfo * jnp.minimum(jnp.maximum(dist16 * inv_dmax, 0.0), 0.9)
                    u = 1.0 - v * v + 1e-8
                    aber = (u * _rsqrt16(u)) / (1.0 + v * dirs16)
                    w_c[pl.ds(g * L, L)] = aber

                def scale(e, _):
                    we = plsc.load_gather(w_c, [jnp.full((L,), e, jnp.int32)])
                    for f in range(H // L):
                        rows[e, pl.ds(f * L, L)] = rows[e, pl.ds(f * L, L)] * we
                    return 0

                lax.fori_loop(0, CH, scale, 0)
                pltpu.sync_copy(rows, acc.at[d_c], add=True)
                return 0

            lax.fori_loop(0, NCHUNK, chunk, 0)
            plsc.subcore_barrier()
            pltpu.sync_copy(acc.at[pl.ds(sid * SL, SL)],
                            part_hbm.at[cid * nobs + o, pl.ds(sid * SL, SL)])
            plsc.subcore_barrier()

    return k(t, src, dst, dist, dirs, maxes, vf16, zrow)


def _finish_body(nobs, sp_ref, aw_ref, ow_ref, ob_ref, o_ref):
    sp = sp_ref[...]
    aw = aw_ref[...]
    hs = [sp[o] + sp[nobs + o] for o in range(nobs)]
    scores = [jnp.sum(jnp.tanh(h) * aw, axis=1, keepdims=True) for h in hs]
    m = scores[0]
    for s in scores[1:]:
        m = jnp.maximum(m, s)
    es = [jnp.exp(s - m) for s in scores]
    den = es[0]
    for e in es[1:]:
        den = den + e
    comb = hs[0] * (es[0] / den)
    for o in range(1, nobs):
        comb = comb + hs[o] * (es[o] / den)
    o_ref[...] = lax.dot_general(
        jnp.maximum(comb, 0.0), ow_ref[...], (((1,), (1,)), ((), ())),
        preferred_element_type=jnp.float32) + ob_ref[...]


def _tc_finish(spart, aw2, out_W, ob2, nobs, N):
    H = spart.shape[2]
    O = out_W.shape[0]
    R = 1000
    return pl.pallas_call(
        functools.partial(_finish_body, nobs),
        grid=(N // R,),
        in_specs=[
            pl.BlockSpec((2 * nobs, R, H), lambda i: (0, i, 0)),
            pl.BlockSpec((1, H), lambda i: (0, 0)),
            pl.BlockSpec((O, H), lambda i: (0, 0)),
            pl.BlockSpec((1, O), lambda i: (0, 0)),
        ],
        out_specs=pl.BlockSpec((R, O), lambda i: (i, 0)),
        out_shape=jax.ShapeDtypeStruct((N, O), jnp.float32),
    )(spart, aw2, out_W, ob2)


def kernel(x, edge_index, W, b, velocity_factors, attn_w, out_W, out_b):
    N, F = x.shape
    H = W.shape[0]
    nobs = velocity_factors.shape[0]
    src = edge_index[0]
    dst = edge_index[1]
    P = min(3, F)
    pos4 = jnp.pad(x[:, :P], ((0, 0), (0, 4 - P))).reshape(-1)
    vf16 = jnp.pad(velocity_factors, (0, L - nobs))
    zrow = jnp.zeros((_slice_rows(N), H), jnp.float32)

    t = _tc_linear(x, W, b.reshape(1, H))
    dist, dirs, maxes = _edge_scalars(pos4, src, dst)
    spart = _accumulate(t, src, dst, dist, dirs, maxes, vf16, zrow, nobs)
    return _tc_finish(spart, attn_w.reshape(1, H), out_W,
                      out_b.reshape(1, out_W.shape[0]), nobs, N)


# speedup vs baseline: 6.2486x; 2.1197x over previous
import functools

import jax
import jax.numpy as jnp
from jax import lax
from jax.experimental import pallas as pl
from jax.experimental.pallas import tpu as pltpu, tpu_sc as plsc

NC = 2
NS = 16
L = 16
NW = NC * NS


def _rsqrt16(s):
    i = lax.bitcast_convert_type(s, jnp.int32)
    i = jnp.int32(0x5F3759DF) - lax.shift_right_arithmetic(i, 1)
    y = lax.bitcast_convert_type(i, jnp.float32)
    for _ in range(3):
        y = y * (1.5 - 0.5 * s * y * y)
    return y


def _full16(v):
    return jnp.full((L,), v, jnp.int32)


def _slice_rows(n):
    return ((n + NS - 1) // NS + 7) // 8 * 8


def _linear_body(x_ref, w_ref, b_ref, o_ref):
    o_ref[...] = lax.dot_general(
        x_ref[...], w_ref[...], (((1,), (1,)), ((), ())),
        preferred_element_type=jnp.float32) + b_ref[...]


def _tc_linear(x, W, b2):
    N, F = x.shape
    H = W.shape[0]
    R = 1000
    return pl.pallas_call(
        _linear_body,
        grid=(N // R,),
        in_specs=[
            pl.BlockSpec((R, F), lambda i: (i, 0)),
            pl.BlockSpec((H, F), lambda i: (0, 0)),
            pl.BlockSpec((1, H), lambda i: (0, 0)),
        ],
        out_specs=pl.BlockSpec((R, H), lambda i: (i, 0)),
        out_shape=jax.ShapeDtypeStruct((N, H), jnp.float32),
    )(x, W, b2)


def _edge_scalars(pos4, src, dst):
    N = pos4.shape[0] // 4
    E = src.shape[0]
    ET = E // NW
    mesh = plsc.VectorSubcoreMesh(core_axis_name="c", subcore_axis_name="s")

    @functools.partial(
        pl.kernel,
        out_type=[
            jax.ShapeDtypeStruct((E,), jnp.float32),
            jax.ShapeDtypeStruct((E,), jnp.float32),
            jax.ShapeDtypeStruct((NW * L,), jnp.float32),
        ],
        mesh=mesh,
        compiler_params=pltpu.CompilerParams(needs_layout_passes=False),
        scratch_types=[
            pltpu.VMEM((N * 4,), jnp.float32),
            pltpu.VMEM((ET,), jnp.int32),
            pltpu.VMEM((ET,), jnp.int32),
            pltpu.VMEM((ET,), jnp.float32),
            pltpu.VMEM((ET,), jnp.float32),
            pltpu.VMEM((L,), jnp.float32),
        ],
    )
    def k(pos_hbm, src_hbm, dst_hbm, dist_hbm, dirs_hbm, max_hbm,
          pos_v, s_v, d_v, dist_v, dirs_v, m_v):
        wid = lax.axis_index("s") * NC + lax.axis_index("c")
        base = wid * ET
        pltpu.sync_copy(pos_hbm, pos_v)
        pltpu.sync_copy(src_hbm.at[pl.ds(base, ET)], s_v)
        pltpu.sync_copy(dst_hbm.at[pl.ds(base, ET)], d_v)

        def body(i, mx):
            s16 = s_v[pl.ds(i * L, L)] * 4
            d16 = d_v[pl.ds(i * L, L)] * 4
            dx = (plsc.load_gather(pos_v, [s16])
                  - plsc.load_gather(pos_v, [d16]))
            dy = (plsc.load_gather(pos_v, [s16 + 1])
                  - plsc.load_gather(pos_v, [d16 + 1]))
            dz = (plsc.load_gather(pos_v, [s16 + 2])
                  - plsc.load_gather(pos_v, [d16 + 2]))
            ss = dx * dx + dy * dy + dz * dz
            sm = jnp.maximum(ss, 1e-30)
            dist = ss * _rsqrt16(sm)
            dirs = (dx + dy + dz) / (dist + 1e-8)
            dist_v[pl.ds(i * L, L)] = dist
            dirs_v[pl.ds(i * L, L)] = dirs
            return jnp.maximum(mx, dist)

        mx = lax.fori_loop(0, ET // L, body, jnp.zeros((L,), jnp.float32))
        m_v[...] = mx
        pltpu.sync_copy(dist_v, dist_hbm.at[pl.ds(base, ET)])
        pltpu.sync_copy(dirs_v, dirs_hbm.at[pl.ds(base, ET)])
        pltpu.sync_copy(m_v, max_hbm.at[pl.ds(wid * L, L)])

    return k(pos4, src, dst)


CH = 80


def _edge_weights(dist, dirs, maxes, vf16, nobs):
    E = dist.shape[0]
    ET = E // NW
    mesh = plsc.VectorSubcoreMesh(core_axis_name="c", subcore_axis_name="s")

    @functools.partial(
        pl.kernel,
        out_type=jax.ShapeDtypeStruct((nobs * E, ), jnp.float32),
        mesh=mesh,
        compiler_params=pltpu.CompilerParams(needs_layout_passes=False),
        scratch_types=[
            pltpu.VMEM((ET,), jnp.float32),
            pltpu.VMEM((ET,), jnp.float32),
            pltpu.VMEM((ET,), jnp.float32),
            pltpu.VMEM((NW * L,), jnp.float32),
            pltpu.VMEM((L,), jnp.float32),
        ],
    )
    def k(dist_hbm, dirs_hbm, max_hbm, vf_hbm, w_hbm,
          di_v, dr_v, w_v, mx_v, vf_v):
        wid = lax.axis_index("s") * NC + lax.axis_index("c")
        base = wid * ET
        pltpu.sync_copy(max_hbm, mx_v)
        pltpu.sync_copy(vf_hbm, vf_v)
        pltpu.sync_copy(dist_hbm.at[pl.ds(base, ET)], di_v)
        pltpu.sync_copy(dirs_hbm.at[pl.ds(base, ET)], dr_v)
        m = jnp.zeros((L,), jnp.float32)
        for j in range(NW):
            m = jnp.maximum(m, mx_v[pl.ds(j * L, L)])
        dmax16 = jnp.full((L,), jnp.max(m, axis=0), jnp.float32)
        inv_dmax = 1.0 / dmax16

        vf_all = vf_v[...]
        lane = lax.iota(jnp.int32, L)
        for o in range(nobs):
            vfo = jnp.full(
                (L,),
                jnp.sum(jnp.where(lane == o, vf_all, 0.0), axis=0),
                jnp.float32)

            def body(g, _):
                dist16 = di_v[pl.ds(g * L, L)]
                dirs16 = dr_v[pl.ds(g * L, L)]
                v = vfo * jnp.minimum(jnp.maximum(dist16 * inv_dmax, 0.0), 0.9)
                u = 1.0 - v * v + 1e-8
                w_v[pl.ds(g * L, L)] = (u * _rsqrt16(u)) / (1.0 + v * dirs16)
                return 0

            lax.fori_loop(0, ET // L, body, 0)
            pltpu.sync_copy(w_v, w_hbm.at[pl.ds(o * E + base, ET)])

    return k(dist, dirs, maxes, vf16)


def _accumulate(t, src, dst, w4, zrow, nobs):
    N, H = t.shape
    E = src.shape[0]
    ET = E // NW
    NCHUNK = ET // CH
    assert NCHUNK % 2 == 1, "pipeline below is unrolled for an odd NCHUNK"
    SL = _slice_rows(N)
    NP = SL * NS
    mesh = plsc.VectorSubcoreMesh(core_axis_name="c", subcore_axis_name="s")

    @functools.partial(
        pl.kernel,
        out_type=jax.ShapeDtypeStruct((NC * nobs, NP, H), jnp.float32),
        mesh=mesh,
        compiler_params=pltpu.CompilerParams(needs_layout_passes=False),
        scratch_types=[
            pltpu.VMEM_SHARED((NP, H), jnp.float32),
            pltpu.VMEM((CH, H), jnp.float32),
            pltpu.VMEM((CH, H), jnp.float32),
            pltpu.VMEM((CH,), jnp.int32),
            pltpu.VMEM((CH,), jnp.int32),
            pltpu.VMEM((CH,), jnp.int32),
            pltpu.VMEM((CH,), jnp.int32),
            pltpu.VMEM((CH,), jnp.float32),
            pltpu.VMEM((CH,), jnp.float32),
            pltpu.SemaphoreType.DMA,
            pltpu.SemaphoreType.DMA,
            pltpu.SemaphoreType.DMA,
            pltpu.SemaphoreType.DMA,
            pltpu.SemaphoreType.DMA,
            pltpu.SemaphoreType.DMA,
        ],
    )
    def k(t_hbm, src_hbm, dst_hbm, w_hbm, z_hbm, part_hbm,
          acc, rows_a, rows_b, s_a, s_b, d_a, d_b, w_a, w_b,
          semr_a, semr_b, semd_a, semd_b, semw_a, semw_b):
        cid = lax.axis_index("c")
        sid = lax.axis_index("s")
        wid = sid * NC + cid
        base = wid * ET
        bufs_a = (s_a, d_a, w_a, rows_a, semr_a, semd_a, semw_a)
        bufs_b = (s_b, d_b, w_b, rows_b, semr_b, semd_b, semw_b)

        def prefetch(i, o, bufs):
            s_c, d_c, w_c, rows, semr, semd, semw = bufs
            cb = base + i * CH
            pltpu.sync_copy(src_hbm.at[pl.ds(cb, CH)], s_c)
            pltpu.async_copy(t_hbm.at[s_c], rows, semr)
            pltpu.async_copy(dst_hbm.at[pl.ds(cb, CH)], d_c, semd)
            pltpu.async_copy(w_hbm.at[pl.ds(o * E + cb, CH)], w_c, semw)

        def process(bufs):
            s_c, d_c, w_c, rows, semr, semd, semw = bufs
            pltpu.make_async_copy(t_hbm.at[s_c], rows, semr).wait()
            pltpu.make_async_copy(dst_hbm.at[pl.ds(0, CH)], d_c, semd).wait()
            pltpu.make_async_copy(w_hbm.at[pl.ds(0, CH)], w_c, semw).wait()

            def scale(e, _):
                we = plsc.load_gather(w_c, [jnp.full((L,), e, jnp.int32)])
                for f in range(H // L):
                    rows[e, pl.ds(f * L, L)] = rows[e, pl.ds(f * L, L)] * we
                return 0

            lax.fori_loop(0, CH, scale, 0)
            pltpu.sync_copy(rows, acc.at[d_c], add=True)

        for o in range(nobs):
            pltpu.sync_copy(z_hbm, acc.at[pl.ds(sid * SL, SL)])
            plsc.subcore_barrier()

            prefetch(0, o, bufs_a)

            def pair(j, _):
                c0 = 2 * j
                prefetch(c0 + 1, o, bufs_b)
                process(bufs_a)
                prefetch(c0 + 2, o, bufs_a)
                process(bufs_b)
                return 0

            lax.fori_loop(0, (NCHUNK - 1) // 2, pair, 0)
            process(bufs_a)
            plsc.subcore_barrier()
            pltpu.sync_copy(acc.at[pl.ds(sid * SL, SL)],
                            part_hbm.at[cid * nobs + o, pl.ds(sid * SL, SL)])
            plsc.subcore_barrier()

    return k(t, src, dst, w4, zrow)


def _finish_body(nobs, sp_ref, aw_ref, ow_ref, ob_ref, o_ref):
    sp = sp_ref[...]
    aw = aw_ref[...]
    hs = [sp[o] + sp[nobs + o] for o in range(nobs)]
    scores = [jnp.sum(jnp.tanh(h) * aw, axis=1, keepdims=True) for h in hs]
    m = scores[0]
    for s in scores[1:]:
        m = jnp.maximum(m, s)
    es = [jnp.exp(s - m) for s in scores]
    den = es[0]
    for e in es[1:]:
        den = den + e
    comb = hs[0] * (es[0] / den)
    for o in range(1, nobs):
        comb = comb + hs[o] * (es[o] / den)
    o_ref[...] = lax.dot_general(
        jnp.maximum(comb, 0.0), ow_ref[...], (((1,), (1,)), ((), ())),
        preferred_element_type=jnp.float32) + ob_ref[...]


def _tc_finish(spart, aw2, out_W, ob2, nobs, N):
    H = spart.shape[2]
    O = out_W.shape[0]
    R = 1000
    return pl.pallas_call(
        functools.partial(_finish_body, nobs),
        grid=(N // R,),
        in_specs=[
            pl.BlockSpec((2 * nobs, R, H), lambda i: (0, i, 0)),
            pl.BlockSpec((1, H), lambda i: (0, 0)),
            pl.BlockSpec((O, H), lambda i: (0, 0)),
            pl.BlockSpec((1, O), lambda i: (0, 0)),
        ],
        out_specs=pl.BlockSpec((R, O), lambda i: (i, 0)),
        out_shape=jax.ShapeDtypeStruct((N, O), jnp.float32),
    )(spart, aw2, out_W, ob2)


def kernel(x, edge_index, W, b, velocity_factors, attn_w, out_W, out_b):
    N, F = x.shape
    H = W.shape[0]
    nobs = velocity_factors.shape[0]
    src = edge_index[0]
    dst = edge_index[1]
    P = min(3, F)
    pos4 = jnp.pad(x[:, :P], ((0, 0), (0, 4 - P))).reshape(-1)
    vf16 = jnp.pad(velocity_factors, (0, L - nobs))
    zrow = jnp.zeros((_slice_rows(N), H), jnp.float32)

    t = _tc_linear(x, W, b.reshape(1, H))
    dist, dirs, maxes = _edge_scalars(pos4, src, dst)
    w4 = _edge_weights(dist, dirs, maxes, vf16, nobs)
    spart = _accumulate(t, src, dst, w4, zrow, nobs)
    return _tc_finish(spart, attn_w.reshape(1, H), out_W,
                      out_b.reshape(1, out_W.shape[0]), nobs, N)


# speedup vs baseline: 6.4838x; 1.0376x over previous
import functools

import jax
import jax.numpy as jnp
from jax import lax
from jax.experimental import pallas as pl
from jax.experimental.pallas import tpu as pltpu, tpu_sc as plsc

NC = 2
NS = 16
L = 16
NW = NC * NS


def _rsqrt16(s):
    i = lax.bitcast_convert_type(s, jnp.int32)
    i = jnp.int32(0x5F3759DF) - lax.shift_right_arithmetic(i, 1)
    y = lax.bitcast_convert_type(i, jnp.float32)
    for _ in range(3):
        y = y * (1.5 - 0.5 * s * y * y)
    return y


def _full16(v):
    return jnp.full((L,), v, jnp.int32)


def _slice_rows(n):
    return ((n + NS - 1) // NS + 7) // 8 * 8


def _linear_body(x_ref, w_ref, b_ref, o_ref):
    o_ref[...] = lax.dot_general(
        x_ref[...], w_ref[...], (((1,), (1,)), ((), ())),
        preferred_element_type=jnp.float32) + b_ref[...]


def _tc_linear(x, W, b2):
    N, F = x.shape
    H = W.shape[0]
    R = 1000
    return pl.pallas_call(
        _linear_body,
        grid=(N // R,),
        in_specs=[
            pl.BlockSpec((R, F), lambda i: (i, 0)),
            pl.BlockSpec((H, F), lambda i: (0, 0)),
            pl.BlockSpec((1, H), lambda i: (0, 0)),
        ],
        out_specs=pl.BlockSpec((R, H), lambda i: (i, 0)),
        out_shape=jax.ShapeDtypeStruct((N, H), jnp.float32),
    )(x, W, b2)


def _edge_scalars(pos4, src, dst):
    N = pos4.shape[0] // 4
    E = src.shape[0]
    ET = E // NW
    mesh = plsc.VectorSubcoreMesh(core_axis_name="c", subcore_axis_name="s")

    @functools.partial(
        pl.kernel,
        out_type=[
            jax.ShapeDtypeStruct((E,), jnp.float32),
            jax.ShapeDtypeStruct((E,), jnp.float32),
            jax.ShapeDtypeStruct((NW * L,), jnp.float32),
        ],
        mesh=mesh,
        compiler_params=pltpu.CompilerParams(needs_layout_passes=False),
        scratch_types=[
            pltpu.VMEM((N * 4,), jnp.float32),
            pltpu.VMEM((ET,), jnp.int32),
            pltpu.VMEM((ET,), jnp.int32),
            pltpu.VMEM((ET,), jnp.float32),
            pltpu.VMEM((ET,), jnp.float32),
            pltpu.VMEM((L,), jnp.float32),
        ],
    )
    def k(pos_hbm, src_hbm, dst_hbm, dist_hbm, dirs_hbm, max_hbm,
          pos_v, s_v, d_v, dist_v, dirs_v, m_v):
        wid = lax.axis_index("s") * NC + lax.axis_index("c")
        base = wid * ET
        pltpu.sync_copy(pos_hbm, pos_v)
        pltpu.sync_copy(src_hbm.at[pl.ds(base, ET)], s_v)
        pltpu.sync_copy(dst_hbm.at[pl.ds(base, ET)], d_v)

        def body(i, mx):
            s16 = s_v[pl.ds(i * L, L)] * 4
            d16 = d_v[pl.ds(i * L, L)] * 4
            dx = (plsc.load_gather(pos_v, [s16])
                  - plsc.load_gather(pos_v, [d16]))
            dy = (plsc.load_gather(pos_v, [s16 + 1])
                  - plsc.load_gather(pos_v, [d16 + 1]))
            dz = (plsc.load_gather(pos_v, [s16 + 2])
                  - plsc.load_gather(pos_v, [d16 + 2]))
            ss = dx * dx + dy * dy + dz * dz
            sm = jnp.maximum(ss, 1e-30)
            dist = ss * _rsqrt16(sm)
            dirs = (dx + dy + dz) / (dist + 1e-8)
            dist_v[pl.ds(i * L, L)] = dist
            dirs_v[pl.ds(i * L, L)] = dirs
            return jnp.maximum(mx, dist)

        mx = lax.fori_loop(0, ET // L, body, jnp.zeros((L,), jnp.float32))
        m_v[...] = mx
        pltpu.sync_copy(dist_v, dist_hbm.at[pl.ds(base, ET)])
        pltpu.sync_copy(dirs_v, dirs_hbm.at[pl.ds(base, ET)])
        pltpu.sync_copy(m_v, max_hbm.at[pl.ds(wid * L, L)])

    return k(pos4, src, dst)


CH = 80


def _edge_weights(dist, dirs, maxes, vf16, nobs):
    E = dist.shape[0]
    ET = E // NW
    mesh = plsc.VectorSubcoreMesh(core_axis_name="c", subcore_axis_name="s")

    @functools.partial(
        pl.kernel,
        out_type=jax.ShapeDtypeStruct((nobs * E, ), jnp.float32),
        mesh=mesh,
        compiler_params=pltpu.CompilerParams(needs_layout_passes=False),
        scratch_types=[
            pltpu.VMEM((ET,), jnp.float32),
            pltpu.VMEM((ET,), jnp.float32),
            pltpu.VMEM((ET,), jnp.float32),
            pltpu.VMEM((NW * L,), jnp.float32),
            pltpu.VMEM((L,), jnp.float32),
        ],
    )
    def k(dist_hbm, dirs_hbm, max_hbm, vf_hbm, w_hbm,
          di_v, dr_v, w_v, mx_v, vf_v):
        wid = lax.axis_index("s") * NC + lax.axis_index("c")
        base = wid * ET
        pltpu.sync_copy(max_hbm, mx_v)
        pltpu.sync_copy(vf_hbm, vf_v)
        pltpu.sync_copy(dist_hbm.at[pl.ds(base, ET)], di_v)
        pltpu.sync_copy(dirs_hbm.at[pl.ds(base, ET)], dr_v)
        m = jnp.zeros((L,), jnp.float32)
        for j in range(NW):
            m = jnp.maximum(m, mx_v[pl.ds(j * L, L)])
        dmax16 = jnp.full((L,), jnp.max(m, axis=0), jnp.float32)
        inv_dmax = 1.0 / dmax16

        vf_all = vf_v[...]
        lane = lax.iota(jnp.int32, L)
        for o in range(nobs):
            vfo = jnp.full(
                (L,),
                jnp.sum(jnp.where(lane == o, vf_all, 0.0), axis=0),
                jnp.float32)

            def body(g, _):
                dist16 = di_v[pl.ds(g * L, L)]
                dirs16 = dr_v[pl.ds(g * L, L)]
                v = vfo * jnp.minimum(jnp.maximum(dist16 * inv_dmax, 0.0), 0.9)
                u = 1.0 - v * v + 1e-8
                w_v[pl.ds(g * L, L)] = (u * _rsqrt16(u)) / (1.0 + v * dirs16)
                return 0

            lax.fori_loop(0, ET // L, body, 0)
            pltpu.sync_copy(w_v, w_hbm.at[pl.ds(o * E + base, ET)])

    return k(dist, dirs, maxes, vf16)


def _accumulate(t, src, dst, w4, zrow, nobs):
    N, H = t.shape
    E = src.shape[0]
    ET = E // NW
    NCHUNK = ET // CH
    assert NCHUNK % 2 == 1, "pipeline below is unrolled for an odd NCHUNK"
    SL = _slice_rows(N)
    NP = SL * NS
    mesh = plsc.VectorSubcoreMesh(core_axis_name="c", subcore_axis_name="s")

    @functools.partial(
        pl.kernel,
        out_type=jax.ShapeDtypeStruct((NC * nobs, NP, H), jnp.float32),
        mesh=mesh,
        compiler_params=pltpu.CompilerParams(needs_layout_passes=False),
        scratch_types=[
            pltpu.VMEM_SHARED((NP, H), jnp.float32),
            pltpu.VMEM((CH, H), jnp.float32),
            pltpu.VMEM((CH, H), jnp.float32),
            pltpu.VMEM((CH,), jnp.int32),
            pltpu.VMEM((CH,), jnp.int32),
            pltpu.VMEM((CH,), jnp.int32),
            pltpu.VMEM((CH,), jnp.int32),
            pltpu.VMEM((CH,), jnp.float32),
            pltpu.VMEM((CH,), jnp.float32),
            pltpu.SemaphoreType.DMA,
            pltpu.SemaphoreType.DMA,
            pltpu.SemaphoreType.DMA,
            pltpu.SemaphoreType.DMA,
            pltpu.SemaphoreType.DMA,
            pltpu.SemaphoreType.DMA,
        ],
    )
    def k(t_hbm, src_hbm, dst_hbm, w_hbm, z_hbm, part_hbm,
          acc, rows_a, rows_b, s_a, s_b, d_a, d_b, w_a, w_b,
          semr_a, semr_b, semd_a, semd_b, semw_a, semw_b):
        cid = lax.axis_index("c")
        sid = lax.axis_index("s")
        wid = sid * NC + cid
        base = wid * ET
        bufs_a = (s_a, d_a, w_a, rows_a, semr_a, semd_a, semw_a)
        bufs_b = (s_b, d_b, w_b, rows_b, semr_b, semd_b, semw_b)

        def prefetch(i, o, bufs):
            s_c, d_c, w_c, rows, semr, semd, semw = bufs
            cb = base + i * CH
            pltpu.sync_copy(src_hbm.at[pl.ds(cb, CH)], s_c)
            pltpu.async_copy(t_hbm.at[s_c], rows, semr)
            pltpu.async_copy(dst_hbm.at[pl.ds(cb, CH)], d_c, semd)
            pltpu.async_copy(w_hbm.at[pl.ds(o * E + cb, CH)], w_c, semw)

        def process(bufs):
            s_c, d_c, w_c, rows, semr, semd, semw = bufs
            pltpu.make_async_copy(t_hbm.at[s_c], rows, semr).wait()
            pltpu.make_async_copy(dst_hbm.at[pl.ds(0, CH)], d_c, semd).wait()
            pltpu.make_async_copy(w_hbm.at[pl.ds(0, CH)], w_c, semw).wait()

            def scale(q, _):
                for u in range(4):
                    e = q * 4 + u
                    we = plsc.load_gather(w_c, [jnp.full((L,), e, jnp.int32)])
                    for f in range(H // L):
                        rows[e, pl.ds(f * L, L)] = rows[e, pl.ds(f * L, L)] * we
                return 0

            lax.fori_loop(0, CH // 4, scale, 0)
            pltpu.sync_copy(rows, acc.at[d_c], add=True)

        for o in range(nobs):
            pltpu.sync_copy(z_hbm, acc.at[pl.ds(sid * SL, SL)])
            plsc.subcore_barrier()

            prefetch(0, o, bufs_a)

            def pair(j, _):
                c0 = 2 * j
                prefetch(c0 + 1, o, bufs_b)
                process(bufs_a)
                prefetch(c0 + 2, o, bufs_a)
                process(bufs_b)
                return 0

            lax.fori_loop(0, (NCHUNK - 1) // 2, pair, 0)
            process(bufs_a)
            plsc.subcore_barrier()
            pltpu.sync_copy(acc.at[pl.ds(sid * SL, SL)],
                            part_hbm.at[cid * nobs + o, pl.ds(sid * SL, SL)])
            plsc.subcore_barrier()

    return k(t, src, dst, w4, zrow)


def _finish_body(nobs, sp_ref, aw_ref, ow_ref, ob_ref, o_ref):
    sp = sp_ref[...]
    aw = aw_ref[...]
    hs = [sp[o] + sp[nobs + o] for o in range(nobs)]
    scores = [jnp.sum(jnp.tanh(h) * aw, axis=1, keepdims=True) for h in hs]
    m = scores[0]
    for s in scores[1:]:
        m = jnp.maximum(m, s)
    es = [jnp.exp(s - m) for s in scores]
    den = es[0]
    for e in es[1:]:
        den = den + e
    comb = hs[0] * (es[0] / den)
    for o in range(1, nobs):
        comb = comb + hs[o] * (es[o] / den)
    o_ref[...] = lax.dot_general(
        jnp.maximum(comb, 0.0), ow_ref[...], (((1,), (1,)), ((), ())),
        preferred_element_type=jnp.float32) + ob_ref[...]


def _tc_finish(spart, aw2, out_W, ob2, nobs, N):
    H = spart.shape[2]
    O = out_W.shape[0]
    R = 1000
    return pl.pallas_call(
        functools.partial(_finish_body, nobs),
        grid=(N // R,),
        in_specs=[
            pl.BlockSpec((2 * nobs, R, H), lambda i: (0, i, 0)),
            pl.BlockSpec((1, H), lambda i: (0, 0)),
            pl.BlockSpec((O, H), lambda i: (0, 0)),
            pl.BlockSpec((1, O), lambda i: (0, 0)),
        ],
        out_specs=pl.BlockSpec((R, O), lambda i: (i, 0)),
        out_shape=jax.ShapeDtypeStruct((N, O), jnp.float32),
    )(spart, aw2, out_W, ob2)


def kernel(x, edge_index, W, b, velocity_factors, attn_w, out_W, out_b):
    N, F = x.shape
    H = W.shape[0]
    nobs = velocity_factors.shape[0]
    src = edge_index[0]
    dst = edge_index[1]
    P = min(3, F)
    pos4 = jnp.pad(x[:, :P], ((0, 0), (0, 4 - P))).reshape(-1)
    vf16 = jnp.pad(velocity_factors, (0, L - nobs))
    zrow = jnp.zeros((_slice_rows(N), H), jnp.float32)

    t = _tc_linear(x, W, b.reshape(1, H))
    dist, dirs, maxes = _edge_scalars(pos4, src, dst)
    w4 = _edge_weights(dist, dirs, maxes, vf16, nobs)
    spart = _accumulate(t, src, dst, w4, zrow, nobs)
    return _tc_finish(spart, attn_w.reshape(1, H), out_W,
                      out_b.reshape(1, out_W.shape[0]), nobs, N)
